# baseline (speedup 1.0000x reference)
import functools
import jax
import jax.numpy as jnp
from jax import lax
from jax.experimental import pallas as pl
from jax.experimental.pallas import tpu as pltpu
from jax.experimental.pallas import tpu_sc as plsc

N = 10000
E = 320000
D = 128
DE = 16
G = 64
H = 32
MSG = 4
OUT = 8

MW = 32
NC = 2
NS = 16
NW = NC * NS
CH = 128
KCH = 80
EPT = KCH * CH
EP = NW * EPT
N4 = N // 4
NSH = N + 48
RPT = NSH // NS

GK = 4
GR = GK * CH
NG = KCH // GK


def _silu(v):
    return v * jax.nn.sigmoid(v)


@functools.cache
def _sc_gather_kernel():
    mesh = plsc.VectorSubcoreMesh(core_axis_name="c", subcore_axis_name="s",
                                  num_cores=NC, num_subcores=NS)

    @functools.partial(
        pl.kernel,
        out_type=(
            jax.ShapeDtypeStruct((EP, H), jnp.bfloat16),
            jax.ShapeDtypeStruct((EP, H), jnp.bfloat16),
            jax.ShapeDtypeStruct((EP, H), jnp.bfloat16),
            jax.ShapeDtypeStruct((EP, H), jnp.bfloat16),
        ),
        mesh=mesh,
        compiler_params=pltpu.CompilerParams(use_tc_tiling_on_sc=False),
        scratch_types=[
            pltpu.VMEM((EPT,), jnp.int32),
            pltpu.VMEM((EPT,), jnp.int32),
            pltpu.VMEM((GR, H), jnp.bfloat16),
            pltpu.VMEM((GR, H), jnp.bfloat16),
            pltpu.VMEM((GR, H), jnp.bfloat16),
            pltpu.VMEM((GR, H), jnp.bfloat16),
            pltpu.SemaphoreType.DMA,
            pltpu.SemaphoreType.DMA,
            pltpu.SemaphoreType.DMA,
            pltpu.SemaphoreType.DMA,
        ],
    )
    def gather(ai_hbm, bi_hbm, ae_hbm, be_hbm, i0i_hbm, i1i_hbm,
               i0e_hbm, i1e_hbm, g0i_hbm, g1i_hbm, g0e_hbm, g1e_hbm,
               i0_v, i1_v, r0a, r0b, r1a, r1b, gs0, gs1, ws0, ws1):
        c = lax.axis_index("c")
        s = lax.axis_index("s")
        wid = c * NS + s
        bufs0 = (r0a, r0b)
        bufs1 = (r1a, r1b)

        for a_hbm, b_hbm, i0_hbm, i1_hbm, g0_hbm, g1_hbm in (
                (ai_hbm, bi_hbm, i0i_hbm, i1i_hbm, g0i_hbm, g1i_hbm),
                (ae_hbm, be_hbm, i0e_hbm, i1e_hbm, g0e_hbm, g1e_hbm)):
            pltpu.sync_copy(i0_hbm.at[wid], i0_v)
            pltpu.sync_copy(i1_hbm.at[wid], i1_v)

            def issue(t, b):
                d0 = pltpu.async_copy(a_hbm.at[i0_v.at[pl.ds(t * GR, GR)]],
                                      bufs0[b], gs0)
                d1 = pltpu.async_copy(b_hbm.at[i1_v.at[pl.ds(t * GR, GR)]],
                                      bufs1[b], gs1)
                return d0, d1

            gd = {0: issue(0, 0)}
            wd = {}
            for t in range(NG):
                b = t % 2
                if t + 1 < NG:
                    if t >= 1:
                        wd[t - 1][0].wait()
                        wd[t - 1][1].wait()
                    gd[t + 1] = issue(t + 1, 1 - b)
                gd[t][0].wait()
                gd[t][1].wait()
                base = wid * EPT + t * GR
                w0 = pltpu.async_copy(bufs0[b], g0_hbm.at[pl.ds(base, GR)], ws0)
                w1 = pltpu.async_copy(bufs1[b], g1_hbm.at[pl.ds(base, GR)], ws1)
                wd[t] = (w0, w1)
            for t in (NG - 2, NG - 1):
                wd[t][0].wait()
                wd[t][1].wait()

    return gather


@functools.cache
def _sc_scatter_kernel():
    mesh = plsc.VectorSubcoreMesh(core_axis_name="c", subcore_axis_name="s",
                                  num_cores=NC, num_subcores=NS)

    @functools.partial(
        pl.kernel,
        out_type=(
            jax.ShapeDtypeStruct((NC, NSH, MW), jnp.float32),
            jax.ShapeDtypeStruct((NC, NSH, MW), jnp.float32),
        ),
        mesh=mesh,
        compiler_params=pltpu.CompilerParams(use_tc_tiling_on_sc=False),
        scratch_types=[
            pltpu.VMEM((KCH, CH), jnp.int32),
            pltpu.VMEM((GR, MW), jnp.float32),
            pltpu.VMEM((GR, MW), jnp.float32),
            pltpu.VMEM((RPT, MW), jnp.float32),
            pltpu.VMEM_SHARED((NSH, MW), jnp.float32),
            pltpu.VMEM_SHARED((NSH, MW), jnp.float32),
            pltpu.SemaphoreType.DMA,
            pltpu.SemaphoreType.DMA,
        ],
    )
    def scatter(mi_hbm, me_hbm, i0i_hbm, i0e_hbm, pi_hbm, pe_hbm,
                i0_v, ra, rb, st_v, acci_sh, acce_sh, ls, ss):
        c = lax.axis_index("c")
        s = lax.axis_index("s")
        wid = c * NS + s

        def zrow(i, carry):
            st_v[i, pl.ds(0, 16)] = jnp.zeros((16,), jnp.float32)
            st_v[i, pl.ds(16, 16)] = jnp.zeros((16,), jnp.float32)
            return carry

        lax.fori_loop(0, RPT, zrow, 0)
        pltpu.sync_copy(st_v, acci_sh.at[pl.ds(s * RPT, RPT)])
        pltpu.sync_copy(st_v, acce_sh.at[pl.ds(s * RPT, RPT)])
        plsc.subcore_barrier()

        bufs = (ra, rb)

        for m_hbm, i0_hbm, acc_sh in ((mi_hbm, i0i_hbm, acci_sh),
                                      (me_hbm, i0e_hbm, acce_sh)):
            pltpu.sync_copy(i0_hbm.at[wid], i0_v)

            def load(t, b):
                return pltpu.async_copy(
                    m_hbm.at[pl.ds(wid * EPT + t * GR, GR)], bufs[b], ls)

            ld = {0: load(0, 0)}
            sd = {}
            for t in range(NG):
                b = t % 2
                if t + 1 < NG:
                    if t >= 1:
                        for k in range(GK):
                            sd[(t - 1, k)].wait()
                    ld[t + 1] = load(t + 1, 1 - b)
                ld[t].wait()
                for k in range(GK):
                    sd[(t, k)] = pltpu.async_copy(
                        bufs[b].at[pl.ds(k * CH, CH)],
                        acc_sh.at[i0_v.at[t * GK + k]], ss, add=True)
            for t in (NG - 2, NG - 1):
                for k in range(GK):
                    sd[(t, k)].wait()
        plsc.subcore_barrier()

        pltpu.sync_copy(acci_sh.at[pl.ds(s * RPT, RPT)], st_v)
        pltpu.sync_copy(st_v, pi_hbm.at[c, pl.ds(s * RPT, RPT)])
        pltpu.sync_copy(acce_sh.at[pl.ds(s * RPT, RPT)], st_v)
        pltpu.sync_copy(st_v, pe_hbm.at[c, pl.ds(s * RPT, RPT)])

    return scatter


def _sc_gather(ai, bi, ae, be, i0i, i1i, i0e, i1e):
    return _sc_gather_kernel()(ai, bi, ae, be, i0i, i1i, i0e, i1e)


def _sc_scatter(mi, me, i0si, i0se):
    return _sc_scatter_kernel()(mi, me, i0si, i0se)


EP8 = EP // 8
E8 = E // 8
NSH4 = NSH // 4
_BQ = 400


def _proj_body(h4_ref, wa_ref, wb_ref, a_ref, b_ref):
    h4 = h4_ref[...]
    a_ref[...] = jnp.dot(h4, wa_ref[...],
                         preferred_element_type=jnp.float32).astype(jnp.bfloat16)
    b_ref[...] = jnp.dot(h4, wb_ref[...],
                         preferred_element_type=jnp.float32).astype(jnp.bfloat16)


def _proj(h4, w4a, w4b):
    return pl.pallas_call(
        _proj_body,
        out_shape=[
            jax.ShapeDtypeStruct((N4, D), jnp.bfloat16),
            jax.ShapeDtypeStruct((N4, D), jnp.bfloat16),
        ],
    )(h4, w4a, w4b)


def _edge_body(g0_ref, g1_ref, ea_ref, w1c_ref, b1_ref, w2_ref, b2_ref, m_ref):
    cc = jnp.dot(ea_ref[...], w1c_ref[...], preferred_element_type=jnp.float32)
    sv = (g0_ref[...].astype(jnp.float32) + g1_ref[...].astype(jnp.float32)
          + cc + b1_ref[...])
    t = _silu(sv)
    m_ref[...] = _silu(jnp.dot(t, w2_ref[...], preferred_element_type=jnp.float32)
                       + b2_ref[...])


def _edge(g0, g1, ea8, w1c8, b1t8, w2bd8, b2t8):
    return pl.pallas_call(
        _edge_body,
        grid=(E8 // _BQ,),
        in_specs=[
            pl.BlockSpec((_BQ, 2 * D), lambda i: (i, 0)),
            pl.BlockSpec((_BQ, 2 * D), lambda i: (i, 0)),
            pl.BlockSpec((_BQ, D), lambda i: (i, 0)),
            pl.BlockSpec((D, 2 * D), lambda i: (0, 0)),
            pl.BlockSpec((1, 2 * D), lambda i: (0, 0)),
            pl.BlockSpec((2 * D, 2 * D), lambda i: (0, 0)),
            pl.BlockSpec((1, 2 * D), lambda i: (0, 0)),
        ],
        out_specs=pl.BlockSpec((_BQ, 2 * D), lambda i: (i, 0)),
        out_shape=jax.ShapeDtypeStruct((EP8, 2 * D), jnp.float32),
    )(g0, g1, ea8, w1c8, b1t8, w2bd8, b2t8)


def _node_body(h4_ref, p_ref, v4a_ref, v4b_ref, b1_ref, w24_ref, b2_ref,
               h_out):
    h4 = h4_ref[...]
    sums4 = p_ref[0, :N4] + p_ref[1, :N4]
    t = _silu(jnp.dot(h4, v4a_ref[...], preferred_element_type=jnp.float32)
              + jnp.dot(sums4, v4b_ref[...], preferred_element_type=jnp.float32)
              + b1_ref[...])
    h_out[...] = _silu(_silu(jnp.dot(t, w24_ref[...],
                                     preferred_element_type=jnp.float32)
                             + b2_ref[...]))


def _node(h4, p4, v4a, v4b, nb1t4, w24, nb2t4):
    return pl.pallas_call(
        _node_body,
        out_shape=jax.ShapeDtypeStruct((N4, 4 * D), jnp.float32),
    )(h4, p4, v4a, v4b, nb1t4, w24, nb2t4)


def _final_body(ui_ref, ue_ref, bt_ref, w1a_ref, w1b_ref, b1_ref,
                w2_ref, b2_ref, o_ref):
    bt = bt_ref[:, :G]
    seg = lax.broadcasted_iota(jnp.int32, (N, G), 1)
    oh = jnp.where(seg == bt, 1.0, 0.0)
    dn = (((0,), (0,)), ((), ()))
    acc_i = lax.dot_general(oh, ui_ref[...], dn,
                            preferred_element_type=jnp.float32)
    acc_e = lax.dot_general(oh, ue_ref[...], dn,
                            preferred_element_type=jnp.float32)
    cnt = lax.dot_general(oh, jnp.ones((N, D), jnp.float32), dn,
                          preferred_element_type=jnp.float32)
    c = jnp.maximum(cnt, 1.0)
    mi = acc_i / c
    me = acc_e / c
    t = _silu(jnp.dot(mi, w1a_ref[...], preferred_element_type=jnp.float32)
              + jnp.dot(me, w1b_ref[...], preferred_element_type=jnp.float32)
              + b1_ref[...])
    o_ref[...] = _silu(jnp.dot(t, w2_ref[...], preferred_element_type=jnp.float32)
                       + b2_ref[...])


def _final(ui, ue, btf, w1a, w1b, b1, w2, b2):
    return pl.pallas_call(
        _final_body,
        out_shape=jax.ShapeDtypeStruct((G, OUT), jnp.float32),
    )(ui, ue, btf, w1a, w1b, b1, w2, b2)


def _kron4(w):
    return jnp.kron(jnp.eye(4, dtype=jnp.float32), w)


def _kron8(w):
    return jnp.kron(jnp.eye(8, dtype=jnp.float32), w)


def _prep_conv(p):
    w1 = p["edge"]["W1"]
    w4a = _kron4(w1[:D])
    w4b = _kron4(w1[D:2 * D])
    w1c8 = _kron8(w1[2 * D:])
    b1t8 = jnp.tile(p["edge"]["b1"].reshape(1, H), (1, 8))
    w2p = jnp.zeros((H, MW), jnp.float32).at[:, :MSG].set(p["edge"]["W2"])
    w2bd8 = _kron8(w2p)
    b2p = jnp.zeros((1, MW), jnp.float32).at[0, :MSG].set(p["edge"]["b2"])
    b2t8 = jnp.tile(b2p, (1, 8))
    nw1 = p["node"]["W1"]
    v4a = _kron4(nw1[:D])
    v1b = jnp.zeros((MW, H), jnp.float32).at[:MSG].set(nw1[D:])
    v4b = _kron4(v1b)
    nb1t4 = jnp.tile(p["node"]["b1"].reshape(1, H), (1, 4))
    w24 = _kron4(p["node"]["W2"])
    nb2t4 = jnp.tile(p["node"]["b2"].reshape(1, D), (1, 4))
    return w4a, w4b, w1c8, b1t8, w2bd8, b2t8, v4a, v4b, nb1t4, w24, nb2t4


def _edge_setup(eidx, eattr):
    pad = EP - E
    n0 = eidx[0].astype(jnp.int32)
    n1 = eidx[1].astype(jnp.int32)
    n0g = jnp.concatenate([n0, jnp.zeros((pad,), jnp.int32)]).reshape(NW, EPT)
    n1g = jnp.concatenate([n1, jnp.zeros((pad,), jnp.int32)]).reshape(NW, EPT)
    n0s = jnp.concatenate([n0, jnp.full((pad,), N, jnp.int32)]).reshape(NW, KCH, CH)
    ea8 = eattr.reshape(E8, D)
    return n0g, n1g, n0s, ea8


def kernel(x, internal_edge_index, internal_edge_attr, edge_index, edge_attr,
           batch, internal_params, external_params, graph_params):
    n0gi, n1gi, n0si, ea8i = _edge_setup(internal_edge_index, internal_edge_attr)
    n0ge, n1ge, n0se, ea8e = _edge_setup(edge_index, edge_attr)
    prep_i = [_prep_conv(p) for p in internal_params]
    prep_e = [_prep_conv(p) for p in external_params]
    h4i = x.reshape(N4, 4 * D)
    h4e = h4i
    for li in range(len(prep_i)):
        pri = prep_i[li]
        pre = prep_e[li]
        ai, bi = _proj(h4i, pri[0], pri[1])
        ae, be = _proj(h4e, pre[0], pre[1])
        g0i, g1i, g0e, g1e = _sc_gather(
            ai.reshape(N, H), bi.reshape(N, H),
            ae.reshape(N, H), be.reshape(N, H), n0gi, n1gi, n0ge, n1ge)
        m8i = _edge(g0i.reshape(EP8, 2 * D), g1i.reshape(EP8, 2 * D),
                    ea8i, pri[2], pri[3], pri[4], pri[5])
        m8e = _edge(g0e.reshape(EP8, 2 * D), g1e.reshape(EP8, 2 * D),
                    ea8e, pre[2], pre[3], pre[4], pre[5])
        pi, pe = _sc_scatter(m8i.reshape(EP, MW), m8e.reshape(EP, MW),
                             n0si, n0se)
        h4i = _node(h4i, pi.reshape(NC, NSH4, D),
                    pri[6], pri[7], pri[8], pri[9], pri[10])
        h4e = _node(h4e, pe.reshape(NC, NSH4, D),
                    pre[6], pre[7], pre[8], pre[9], pre[10])
    upd_int = h4i.reshape(N, D)
    upd_ext = h4e.reshape(N, D)
    btf = jnp.broadcast_to(batch.astype(jnp.int32)[:, None], (N, D))
    gw1 = graph_params["W1"]
    out = _final(upd_int, upd_ext, btf,
                 gw1[:D], gw1[D:], graph_params["b1"].reshape(1, H),
                 graph_params["W2"], graph_params["b2"].reshape(1, OUT))
    return out


# speedup vs baseline: 1.0548x; 1.0548x over previous
import functools
import jax
import jax.numpy as jnp
from jax import lax
from jax.experimental import pallas as pl
from jax.experimental.pallas import tpu as pltpu
from jax.experimental.pallas import tpu_sc as plsc

N = 10000
E = 320000
D = 128
DE = 16
G = 64
H = 32
MSG = 4
OUT = 8

MW = 32
NC = 2
NS = 16
NW = NC * NS
CH = 128
KCH = 80
EPT = KCH * CH
EP = NW * EPT
N4 = N // 4
NSH = N + 48
RPT = NSH // NS

GK = 4
GR = GK * CH
NG = KCH // GK


def _silu(v):
    return v * jax.nn.sigmoid(v)


@functools.cache
def _sc_gather_kernel():
    mesh = plsc.VectorSubcoreMesh(core_axis_name="c", subcore_axis_name="s",
                                  num_cores=NC, num_subcores=NS)

    @functools.partial(
        pl.kernel,
        out_type=(
            jax.ShapeDtypeStruct((EP, H), jnp.bfloat16),
            jax.ShapeDtypeStruct((EP, H), jnp.bfloat16),
        ),
        mesh=mesh,
        compiler_params=pltpu.CompilerParams(use_tc_tiling_on_sc=False),
        scratch_types=[
            pltpu.VMEM((EPT,), jnp.int32),
            pltpu.VMEM((EPT,), jnp.int32),
            pltpu.VMEM((GR, H), jnp.bfloat16),
            pltpu.VMEM((GR, H), jnp.bfloat16),
            pltpu.VMEM((GR, H), jnp.bfloat16),
            pltpu.VMEM((GR, H), jnp.bfloat16),
            pltpu.SemaphoreType.DMA,
            pltpu.SemaphoreType.DMA,
            pltpu.SemaphoreType.DMA,
            pltpu.SemaphoreType.DMA,
        ],
    )
    def gather(a_hbm, b_hbm, i0_hbm, i1_hbm, g0_hbm, g1_hbm,
               i0_v, i1_v, r0a, r0b, r1a, r1b, gs0, gs1, ws0, ws1):
        c = lax.axis_index("c")
        s = lax.axis_index("s")
        wid = c * NS + s
        pltpu.sync_copy(i0_hbm.at[wid], i0_v)
        pltpu.sync_copy(i1_hbm.at[wid], i1_v)
        bufs0 = (r0a, r0b)
        bufs1 = (r1a, r1b)

        def issue(t, b):
            d0 = pltpu.async_copy(a_hbm.at[i0_v.at[pl.ds(t * GR, GR)]],
                                  bufs0[b], gs0)
            d1 = pltpu.async_copy(b_hbm.at[i1_v.at[pl.ds(t * GR, GR)]],
                                  bufs1[b], gs1)
            return d0, d1

        gd = {0: issue(0, 0)}
        wd = {}
        for t in range(NG):
            b = t % 2
            if t + 1 < NG:
                if t >= 1:
                    wd[t - 1][0].wait()
                    wd[t - 1][1].wait()
                gd[t + 1] = issue(t + 1, 1 - b)
            gd[t][0].wait()
            gd[t][1].wait()
            base = wid * EPT + t * GR
            w0 = pltpu.async_copy(bufs0[b], g0_hbm.at[pl.ds(base, GR)], ws0)
            w1 = pltpu.async_copy(bufs1[b], g1_hbm.at[pl.ds(base, GR)], ws1)
            wd[t] = (w0, w1)
        for t in (NG - 2, NG - 1):
            wd[t][0].wait()
            wd[t][1].wait()

    return gather


@functools.cache
def _sc_scatter_kernel():
    mesh = plsc.VectorSubcoreMesh(core_axis_name="c", subcore_axis_name="s",
                                  num_cores=NC, num_subcores=NS)

    @functools.partial(
        pl.kernel,
        out_type=jax.ShapeDtypeStruct((NC, NSH, MW), jnp.float32),
        mesh=mesh,
        compiler_params=pltpu.CompilerParams(use_tc_tiling_on_sc=False),
        scratch_types=[
            pltpu.VMEM((KCH, CH), jnp.int32),
            pltpu.VMEM((GR, MW), jnp.float32),
            pltpu.VMEM((GR, MW), jnp.float32),
            pltpu.VMEM((RPT, MW), jnp.float32),
            pltpu.VMEM_SHARED((NSH, MW), jnp.float32),
            pltpu.SemaphoreType.DMA,
            pltpu.SemaphoreType.DMA,
        ],
    )
    def scatter(m_hbm, i0_hbm, p_hbm, i0_v, ra, rb, st_v, acc_sh, ls, ss):
        c = lax.axis_index("c")
        s = lax.axis_index("s")
        wid = c * NS + s

        def zrow(i, carry):
            st_v[i, pl.ds(0, 16)] = jnp.zeros((16,), jnp.float32)
            st_v[i, pl.ds(16, 16)] = jnp.zeros((16,), jnp.float32)
            return carry

        lax.fori_loop(0, RPT, zrow, 0)
        pltpu.sync_copy(i0_hbm.at[wid], i0_v)
        pltpu.sync_copy(st_v, acc_sh.at[pl.ds(s * RPT, RPT)])
        plsc.subcore_barrier()

        bufs = (ra, rb)

        def load(t, b):
            return pltpu.async_copy(
                m_hbm.at[pl.ds(wid * EPT + t * GR, GR)], bufs[b], ls)

        ld = {0: load(0, 0)}
        sd = {}
        for t in range(NG):
            b = t % 2
            if t + 1 < NG:
                if t >= 1:
                    for k in range(GK):
                        sd[(t - 1, k)].wait()
                ld[t + 1] = load(t + 1, 1 - b)
            ld[t].wait()
            for k in range(GK):
                sd[(t, k)] = pltpu.async_copy(
                    bufs[b].at[pl.ds(k * CH, CH)],
                    acc_sh.at[i0_v.at[t * GK + k]], ss, add=True)
        for t in (NG - 2, NG - 1):
            for k in range(GK):
                sd[(t, k)].wait()
        plsc.subcore_barrier()

        pltpu.sync_copy(acc_sh.at[pl.ds(s * RPT, RPT)], st_v)
        pltpu.sync_copy(st_v, p_hbm.at[c, pl.ds(s * RPT, RPT)])

    return scatter


def _sc_gather(a, b, i0, i1):
    return _sc_gather_kernel()(a, b, i0, i1)


def _sc_scatter(m, i0s):
    return _sc_scatter_kernel()(m, i0s)


EP8 = EP // 8
E8 = E // 8
NSH4 = NSH // 4
_BQ = 400


def _proj_body(h4_ref, wa_ref, wb_ref, a_ref, b_ref):
    h4 = h4_ref[...]
    a_ref[...] = jnp.dot(h4, wa_ref[...],
                         preferred_element_type=jnp.float32).astype(jnp.bfloat16)
    b_ref[...] = jnp.dot(h4, wb_ref[...],
                         preferred_element_type=jnp.float32).astype(jnp.bfloat16)


def _proj(h4, w4a, w4b):
    return pl.pallas_call(
        _proj_body,
        out_shape=[
            jax.ShapeDtypeStruct((N4, D), jnp.bfloat16),
            jax.ShapeDtypeStruct((N4, D), jnp.bfloat16),
        ],
    )(h4, w4a, w4b)


def _edge_body(g0_ref, g1_ref, ea_ref, w1c_ref, b1_ref, w2_ref, b2_ref, m_ref):
    cc = jnp.dot(ea_ref[...], w1c_ref[...], preferred_element_type=jnp.float32)
    sv = (g0_ref[...].astype(jnp.float32) + g1_ref[...].astype(jnp.float32)
          + cc + b1_ref[...])
    t = _silu(sv)
    m_ref[...] = _silu(jnp.dot(t, w2_ref[...], preferred_element_type=jnp.float32)
                       + b2_ref[...])


def _edge(g0, g1, ea8, w1c8, b1t8, w2bd8, b2t8):
    return pl.pallas_call(
        _edge_body,
        grid=(E8 // _BQ,),
        in_specs=[
            pl.BlockSpec((_BQ, 2 * D), lambda i: (i, 0)),
            pl.BlockSpec((_BQ, 2 * D), lambda i: (i, 0)),
            pl.BlockSpec((_BQ, D), lambda i: (i, 0)),
            pl.BlockSpec((D, 2 * D), lambda i: (0, 0)),
            pl.BlockSpec((1, 2 * D), lambda i: (0, 0)),
            pl.BlockSpec((2 * D, 2 * D), lambda i: (0, 0)),
            pl.BlockSpec((1, 2 * D), lambda i: (0, 0)),
        ],
        out_specs=pl.BlockSpec((_BQ, 2 * D), lambda i: (i, 0)),
        out_shape=jax.ShapeDtypeStruct((EP8, 2 * D), jnp.float32),
    )(g0, g1, ea8, w1c8, b1t8, w2bd8, b2t8)


def _node_body(h4_ref, p_ref, v4a_ref, v4b_ref, b1_ref, w24_ref, b2_ref,
               h_out):
    h4 = h4_ref[...]
    sums4 = p_ref[0, :N4] + p_ref[1, :N4]
    t = _silu(jnp.dot(h4, v4a_ref[...], preferred_element_type=jnp.float32)
              + jnp.dot(sums4, v4b_ref[...], preferred_element_type=jnp.float32)
              + b1_ref[...])
    h_out[...] = _silu(_silu(jnp.dot(t, w24_ref[...],
                                     preferred_element_type=jnp.float32)
                             + b2_ref[...]))


def _node(h4, p4, v4a, v4b, nb1t4, w24, nb2t4):
    return pl.pallas_call(
        _node_body,
        out_shape=jax.ShapeDtypeStruct((N4, 4 * D), jnp.float32),
    )(h4, p4, v4a, v4b, nb1t4, w24, nb2t4)


def _final_body(ui_ref, ue_ref, bt_ref, w1a_ref, w1b_ref, b1_ref,
                w2_ref, b2_ref, o_ref):
    bt = bt_ref[:, :G]
    seg = lax.broadcasted_iota(jnp.int32, (N, G), 1)
    oh = jnp.where(seg == bt, 1.0, 0.0)
    dn = (((0,), (0,)), ((), ()))
    acc_i = lax.dot_general(oh, ui_ref[...], dn,
                            preferred_element_type=jnp.float32)
    acc_e = lax.dot_general(oh, ue_ref[...], dn,
                            preferred_element_type=jnp.float32)
    cnt = lax.dot_general(oh, jnp.ones((N, D), jnp.float32), dn,
                          preferred_element_type=jnp.float32)
    c = jnp.maximum(cnt, 1.0)
    mi = acc_i / c
    me = acc_e / c
    t = _silu(jnp.dot(mi, w1a_ref[...], preferred_element_type=jnp.float32)
              + jnp.dot(me, w1b_ref[...], preferred_element_type=jnp.float32)
              + b1_ref[...])
    o_ref[...] = _silu(jnp.dot(t, w2_ref[...], preferred_element_type=jnp.float32)
                       + b2_ref[...])


def _final(ui, ue, btf, w1a, w1b, b1, w2, b2):
    return pl.pallas_call(
        _final_body,
        out_shape=jax.ShapeDtypeStruct((G, OUT), jnp.float32),
    )(ui, ue, btf, w1a, w1b, b1, w2, b2)


def _kron4(w):
    return jnp.kron(jnp.eye(4, dtype=jnp.float32), w)


def _kron8(w):
    return jnp.kron(jnp.eye(8, dtype=jnp.float32), w)


def _prep_conv(p):
    w1 = p["edge"]["W1"]
    w4a = _kron4(w1[:D])
    w4b = _kron4(w1[D:2 * D])
    w1c8 = _kron8(w1[2 * D:])
    b1t8 = jnp.tile(p["edge"]["b1"].reshape(1, H), (1, 8))
    w2p = jnp.zeros((H, MW), jnp.float32).at[:, :MSG].set(p["edge"]["W2"])
    w2bd8 = _kron8(w2p)
    b2p = jnp.zeros((1, MW), jnp.float32).at[0, :MSG].set(p["edge"]["b2"])
    b2t8 = jnp.tile(b2p, (1, 8))
    nw1 = p["node"]["W1"]
    v4a = _kron4(nw1[:D])
    v1b = jnp.zeros((MW, H), jnp.float32).at[:MSG].set(nw1[D:])
    v4b = _kron4(v1b)
    nb1t4 = jnp.tile(p["node"]["b1"].reshape(1, H), (1, 4))
    w24 = _kron4(p["node"]["W2"])
    nb2t4 = jnp.tile(p["node"]["b2"].reshape(1, D), (1, 4))
    return w4a, w4b, w1c8, b1t8, w2bd8, b2t8, v4a, v4b, nb1t4, w24, nb2t4


def _edge_setup(eidx, eattr):
    pad = EP - E
    n0 = eidx[0].astype(jnp.int32)
    n1 = eidx[1].astype(jnp.int32)
    n0g = jnp.concatenate([n0, jnp.zeros((pad,), jnp.int32)]).reshape(NW, EPT)
    n1g = jnp.concatenate([n1, jnp.zeros((pad,), jnp.int32)]).reshape(NW, EPT)
    n0s = jnp.concatenate([n0, jnp.full((pad,), N, jnp.int32)]).reshape(NW, KCH, CH)
    ea8 = eattr.reshape(E8, D)
    return n0g, n1g, n0s, ea8


def _layer(h4, setup, pr):
    n0g, n1g, n0s, ea8 = setup
    w4a, w4b, w1c8, b1t8, w2bd8, b2t8, v4a, v4b, nb1t4, w24, nb2t4 = pr
    a, b = _proj(h4, w4a, w4b)
    g0, g1 = _sc_gather(a.reshape(N, H), b.reshape(N, H), n0g, n1g)
    m8 = _edge(g0.reshape(EP8, 2 * D), g1.reshape(EP8, 2 * D),
               ea8, w1c8, b1t8, w2bd8, b2t8)
    psum = _sc_scatter(m8.reshape(EP, MW), n0s)
    return _node(h4, psum.reshape(NC, NSH4, D), v4a, v4b, nb1t4, w24, nb2t4)


def kernel(x, internal_edge_index, internal_edge_attr, edge_index, edge_attr,
           batch, internal_params, external_params, graph_params):
    setup_i = _edge_setup(internal_edge_index, internal_edge_attr)
    setup_e = _edge_setup(edge_index, edge_attr)
    prep_i = [_prep_conv(p) for p in internal_params]
    prep_e = [_prep_conv(p) for p in external_params]
    h4i = x.reshape(N4, 4 * D)
    h4e = h4i
    for li in range(len(prep_i)):
        h4i = _layer(h4i, setup_i, prep_i[li])
        h4e = _layer(h4e, setup_e, prep_e[li])
    upd_int = h4i.reshape(N, D)
    upd_ext = h4e.reshape(N, D)
    btf = jnp.broadcast_to(batch.astype(jnp.int32)[:, None], (N, D))
    gw1 = graph_params["W1"]
    out = _final(upd_int, upd_ext, btf,
                 gw1[:D], gw1[D:], graph_params["b1"].reshape(1, H),
                 graph_params["W2"], graph_params["b2"].reshape(1, OUT))
    return out
